# baseline (speedup 1.0000x reference)
import functools

import jax
import jax.numpy as jnp
from jax.experimental import pallas as pl

S, D, H, DH = 8192, 768, 12, 64
NLAYERS, R, NB, DFF = 2, 2, 64, 3072
CHUNK = 128
NC = S // CHUNK
RH = R * H
SB = 512
FSB = 256


def _proj_body(x_ref, g_ref, b_ref, wqk_ref, wv_ref, qk_ref, v_ref):
    x = x_ref[...]
    m = jnp.mean(x, axis=-1, keepdims=True)
    var = jnp.mean((x - m) ** 2, axis=-1, keepdims=True)
    n = (x - m) * jax.lax.rsqrt(var + 1e-5) * g_ref[...] + b_ref[...]
    qk_ref[...] = jnp.dot(n, wqk_ref[...], preferred_element_type=jnp.float32)
    v_ref[...] = jnp.dot(n, wv_ref[...], preferred_element_type=jnp.float32)


def _proj(x, g, b, wqk, wv):
    nsb = S // SB
    return pl.pallas_call(
        _proj_body,
        grid=(nsb,),
        in_specs=[
            pl.BlockSpec((SB, D), lambda i: (i, 0)),
            pl.BlockSpec((1, D), lambda i: (0, 0)),
            pl.BlockSpec((1, D), lambda i: (0, 0)),
            pl.BlockSpec((D, D), lambda i: (0, 0)),
            pl.BlockSpec((D, D), lambda i: (0, 0)),
        ],
        out_specs=[
            pl.BlockSpec((SB, D), lambda i: (i, 0)),
            pl.BlockSpec((SB, D), lambda i: (i, 0)),
        ],
        out_shape=[
            jax.ShapeDtypeStruct((S, D), jnp.float32),
            jax.ShapeDtypeStruct((S, D), jnp.float32),
        ],
    )(x, g.reshape(1, D), b.reshape(1, D), wqk, wv)


NCP = 16
SEG = NCP * CHUNK


def _attn_one_chunk(q, kp, vc, vp, pq, pp, o_ref, lse_ref, k):
    k2 = jnp.concatenate([kp, q], axis=0)
    v2 = jnp.concatenate([vp, vc], axis=0)
    norm = jnp.sqrt(jnp.sum(k2 * k2, axis=-1, keepdims=True))
    k2n = k2 / jnp.maximum(norm, 1e-6)
    dots = jax.lax.dot_general(
        q, k2n, (((1,), (1,)), ((), ())),
        preferred_element_type=jnp.float32) * (DH ** -0.5)
    pk = jnp.concatenate([pp, pq], axis=1)
    pqc = jnp.transpose(pq)
    dots = jnp.where(pqc >= pk, dots, -1e9)
    dots = jnp.where(pqc == pk, dots - 1e5, dots)
    m = jnp.max(dots, axis=-1, keepdims=True)
    ex = jnp.exp(dots - m)
    ssum = jnp.sum(ex, axis=-1, keepdims=True)
    o_ref[0, k * CHUNK:(k + 1) * CHUNK, :] = jnp.dot(
        ex / ssum, v2, preferred_element_type=jnp.float32)
    lse_ref[0, k:k + 1, :] = jnp.transpose(m + jnp.log(ssum))


def _attn_body(cur_ref, prevb_ref, vcur_ref, vprevb_ref, pcur_ref, pprevb_ref,
               o_ref, lse_ref):
    for k in range(NCP):
        q = cur_ref[0, k * CHUNK:(k + 1) * CHUNK, :]
        vc = vcur_ref[0, k * CHUNK:(k + 1) * CHUNK, :]
        if k == 0:
            kp, vp, pp = prevb_ref[0], vprevb_ref[0], pprevb_ref[0, 0]
        else:
            kp = cur_ref[0, (k - 1) * CHUNK:k * CHUNK, :]
            vp = vcur_ref[0, (k - 1) * CHUNK:k * CHUNK, :]
            pp = pcur_ref[0, k - 1, :, :]
        pq = pcur_ref[0, k, :, :]
        _attn_one_chunk(q, kp, vc, vp, pq, pp, o_ref, lse_ref, k)


def _attention(sqk, sv, spos):
    prevc = lambda cb: jax.lax.rem(cb * NCP + NC - 1, NC)
    return pl.pallas_call(
        _attn_body,
        grid=(RH, S // SEG),
        in_specs=[
            pl.BlockSpec((1, SEG, DH), lambda j, cb: (j, cb, 0)),
            pl.BlockSpec((1, CHUNK, DH), lambda j, cb: (j, prevc(cb), 0)),
            pl.BlockSpec((1, SEG, DH), lambda j, cb: (j, cb, 0)),
            pl.BlockSpec((1, CHUNK, DH), lambda j, cb: (j, prevc(cb), 0)),
            pl.BlockSpec((1, NCP, 1, CHUNK), lambda j, cb: (j, cb, 0, 0)),
            pl.BlockSpec((1, 1, 1, CHUNK), lambda j, cb: (j, prevc(cb), 0, 0)),
        ],
        out_specs=[
            pl.BlockSpec((1, SEG, DH), lambda j, cb: (j, cb, 0)),
            pl.BlockSpec((1, NCP, CHUNK), lambda j, cb: (j, cb, 0)),
        ],
        out_shape=[
            jax.ShapeDtypeStruct((RH, S, DH), jnp.float32),
            jax.ShapeDtypeStruct((RH, NC, CHUNK), jnp.float32),
        ],
    )(sqk, sqk, sv, sv, spos, spos)


def _comb_body(o0_ref, o1_ref, w0_ref, x1_ref, wo_ref, y1_ref):
    w0 = w0_ref[...]
    oc = o0_ref[...] * w0 + o1_ref[...] * (1.0 - w0)
    y1_ref[...] = x1_ref[...] + jnp.dot(
        oc, wo_ref[...], preferred_element_type=jnp.float32)


def _combine(o0, o1, w0, x1, wo):
    nsb = S // SB
    return pl.pallas_call(
        _comb_body,
        grid=(nsb,),
        in_specs=[
            pl.BlockSpec((SB, D), lambda i: (i, 0)),
            pl.BlockSpec((SB, D), lambda i: (i, 0)),
            pl.BlockSpec((SB, D), lambda i: (i, 0)),
            pl.BlockSpec((SB, D), lambda i: (i, 0)),
            pl.BlockSpec((D, D), lambda i: (0, 0)),
        ],
        out_specs=pl.BlockSpec((SB, D), lambda i: (i, 0)),
        out_shape=jax.ShapeDtypeStruct((S, D), jnp.float32),
    )(o0, o1, w0, x1, wo)


def _ffn_body(y1_ref, x2_ref, g_ref, b_ref, w1_ref, b1_ref, w2_ref, b2_ref,
              y2_ref):
    x = y1_ref[...]
    m = jnp.mean(x, axis=-1, keepdims=True)
    var = jnp.mean((x - m) ** 2, axis=-1, keepdims=True)
    n2 = (x - m) * jax.lax.rsqrt(var + 1e-5) * g_ref[...] + b_ref[...]
    h = jnp.dot(n2, w1_ref[...], preferred_element_type=jnp.float32) + b1_ref[...]
    h = jax.nn.gelu(h)
    y2_ref[...] = x2_ref[...] + jnp.dot(
        h, w2_ref[...], preferred_element_type=jnp.float32) + b2_ref[...]


def _ffn(y1, x2, g, b, w1, b1, w2, b2):
    nsb = S // FSB
    return pl.pallas_call(
        _ffn_body,
        grid=(nsb,),
        in_specs=[
            pl.BlockSpec((FSB, D), lambda i: (i, 0)),
            pl.BlockSpec((FSB, D), lambda i: (i, 0)),
            pl.BlockSpec((1, D), lambda i: (0, 0)),
            pl.BlockSpec((1, D), lambda i: (0, 0)),
            pl.BlockSpec((D, DFF), lambda i: (0, 0)),
            pl.BlockSpec((1, DFF), lambda i: (0, 0)),
            pl.BlockSpec((DFF, D), lambda i: (0, 0)),
            pl.BlockSpec((1, D), lambda i: (0, 0)),
        ],
        out_specs=pl.BlockSpec((FSB, D), lambda i: (i, 0)),
        out_shape=jax.ShapeDtypeStruct((S, D), jnp.float32),
    )(y1, x2, g.reshape(1, D), b.reshape(1, D), w1, b1.reshape(1, DFF), w2,
      b2.reshape(1, D))


def kernel(x1, x2, Wqk, Wv, Wo, ln1_g, ln1_b, ln2_g, ln2_b, W1, b1, W2, b2,
           rot):
    x1 = x1[0]
    x2 = x2[0]
    pos = jnp.arange(S, dtype=jnp.int32)
    for i in range(NLAYERS):
        qk, vv = _proj(x2, ln1_g[i], ln1_b[i], Wqk[i], Wv[i])
        qkh = qk.reshape(S, H, DH).transpose(1, 0, 2)
        vvh = vv.reshape(S, H, DH).transpose(1, 0, 2)
        rotated = jnp.einsum('hsd,drn->hsrn', qkh, rot[i])
        buckets = jnp.argmax(
            jnp.concatenate([rotated, -rotated], axis=-1), axis=-1)
        bkt = buckets.astype(jnp.int32).transpose(2, 0, 1)
        skey = bkt * S + pos[None, None, :]
        sidx = jnp.argsort(skey, axis=-1).astype(jnp.int32)
        sidx_f = sidx.reshape(RH, S)
        rowix = jnp.arange(RH, dtype=jnp.int32)[:, None]
        qkh2 = jnp.broadcast_to(qkh[None], (R, H, S, DH)).reshape(RH, S, DH)
        vvh2 = jnp.broadcast_to(vvh[None], (R, H, S, DH)).reshape(RH, S, DH)
        sqk = jnp.take_along_axis(qkh2, sidx_f[..., None], axis=1)
        sv = jnp.take_along_axis(vvh2, sidx_f[..., None], axis=1)
        spos = sidx_f.reshape(RH, NC, 1, CHUNK)
        o_s, lse_s = _attention(sqk, sv, spos)
        o_u = jnp.zeros((RH, S, DH), jnp.float32).at[rowix, sidx_f].set(o_s)
        lse_u = jnp.zeros((RH, S), jnp.float32).at[rowix, sidx_f].set(
            lse_s.reshape(RH, S))
        o_u = o_u.reshape(R, H, S, DH).transpose(0, 2, 1, 3).reshape(R, S, D)
        lse_u = lse_u.reshape(R, H, S).transpose(0, 2, 1)
        w = jax.nn.softmax(lse_u, axis=0)
        w0 = jnp.repeat(w[0], DH, axis=-1)
        y1 = _combine(o_u[0], o_u[1], w0, x1, Wo[i])
        y2 = _ffn(y1, x2, ln2_g[i], ln2_b[i], W1[i], b1[i], W2[i], b2[i])
        x1, x2 = y1, y2
    return x2[None]


# speedup vs baseline: 1.6403x; 1.6403x over previous
import functools

import jax
import jax.numpy as jnp
from jax.experimental import pallas as pl

S, D, H, DH = 8192, 768, 12, 64
NLAYERS, R, NB, DFF = 2, 2, 64, 3072
CHUNK = 128
NC = S // CHUNK
RH = R * H
SB = 512
FSB = 256


def _proj_body(x_ref, g_ref, b_ref, wqk_ref, wv_ref, qk_ref, v_ref):
    x = x_ref[...]
    m = jnp.mean(x, axis=-1, keepdims=True)
    var = jnp.mean((x - m) ** 2, axis=-1, keepdims=True)
    n = (x - m) * jax.lax.rsqrt(var + 1e-5) * g_ref[...] + b_ref[...]
    qk_ref[...] = jnp.dot(n, wqk_ref[...], preferred_element_type=jnp.float32)
    v_ref[...] = jnp.dot(n, wv_ref[...], preferred_element_type=jnp.float32)


def _proj(x, g, b, wqk, wv):
    nsb = S // SB
    return pl.pallas_call(
        _proj_body,
        grid=(nsb,),
        in_specs=[
            pl.BlockSpec((SB, D), lambda i: (i, 0)),
            pl.BlockSpec((1, D), lambda i: (0, 0)),
            pl.BlockSpec((1, D), lambda i: (0, 0)),
            pl.BlockSpec((D, D), lambda i: (0, 0)),
            pl.BlockSpec((D, D), lambda i: (0, 0)),
        ],
        out_specs=[
            pl.BlockSpec((SB, D), lambda i: (i, 0)),
            pl.BlockSpec((SB, D), lambda i: (i, 0)),
        ],
        out_shape=[
            jax.ShapeDtypeStruct((S, D), jnp.float32),
            jax.ShapeDtypeStruct((S, D), jnp.float32),
        ],
    )(x, g.reshape(1, D), b.reshape(1, D), wqk, wv)


NCP = 16
SEG = NCP * CHUNK


def _attn_one_chunk(q, kp, vc, vp, pq, pp, o_ref, lse_ref, k):
    k2 = jnp.concatenate([kp, q], axis=0)
    v2 = jnp.concatenate([vp, vc], axis=0)
    norm = jnp.sqrt(jnp.sum(k2 * k2, axis=-1, keepdims=True))
    k2n = k2 / jnp.maximum(norm, 1e-6)
    dots = jax.lax.dot_general(
        q, k2n, (((1,), (1,)), ((), ())),
        preferred_element_type=jnp.float32) * (DH ** -0.5)
    pk = jnp.concatenate([pp, pq], axis=1)
    pqc = jnp.transpose(pq)
    dots = jnp.where(pqc >= pk, dots, -1e9)
    dots = jnp.where(pqc == pk, dots - 1e5, dots)
    m = jnp.max(dots, axis=-1, keepdims=True)
    ex = jnp.exp(dots - m)
    ssum = jnp.sum(ex, axis=-1, keepdims=True)
    o_ref[0, k * CHUNK:(k + 1) * CHUNK, :] = jnp.dot(
        ex / ssum, v2, preferred_element_type=jnp.float32)
    lse_ref[0, k:k + 1, :] = jnp.transpose(m + jnp.log(ssum))


def _attn_body(cur_ref, prevb_ref, vcur_ref, vprevb_ref, pcur_ref, pprevb_ref,
               o_ref, lse_ref):
    for k in range(NCP):
        q = cur_ref[0, k * CHUNK:(k + 1) * CHUNK, :]
        vc = vcur_ref[0, k * CHUNK:(k + 1) * CHUNK, :]
        if k == 0:
            kp, vp, pp = prevb_ref[0], vprevb_ref[0], pprevb_ref[0, 0]
        else:
            kp = cur_ref[0, (k - 1) * CHUNK:k * CHUNK, :]
            vp = vcur_ref[0, (k - 1) * CHUNK:k * CHUNK, :]
            pp = pcur_ref[0, k - 1, :, :]
        pq = pcur_ref[0, k, :, :]
        _attn_one_chunk(q, kp, vc, vp, pq, pp, o_ref, lse_ref, k)


def _attention(sqk, sv, spos):
    prevc = lambda cb: jax.lax.rem(cb * NCP + NC - 1, NC)
    return pl.pallas_call(
        _attn_body,
        grid=(RH, S // SEG),
        in_specs=[
            pl.BlockSpec((1, SEG, DH), lambda j, cb: (j, cb, 0)),
            pl.BlockSpec((1, CHUNK, DH), lambda j, cb: (j, prevc(cb), 0)),
            pl.BlockSpec((1, SEG, DH), lambda j, cb: (j, cb, 0)),
            pl.BlockSpec((1, CHUNK, DH), lambda j, cb: (j, prevc(cb), 0)),
            pl.BlockSpec((1, NCP, 1, CHUNK), lambda j, cb: (j, cb, 0, 0)),
            pl.BlockSpec((1, 1, 1, CHUNK), lambda j, cb: (j, prevc(cb), 0, 0)),
        ],
        out_specs=[
            pl.BlockSpec((1, SEG, DH), lambda j, cb: (j, cb, 0)),
            pl.BlockSpec((1, NCP, CHUNK), lambda j, cb: (j, cb, 0)),
        ],
        out_shape=[
            jax.ShapeDtypeStruct((RH, S, DH), jnp.float32),
            jax.ShapeDtypeStruct((RH, NC, CHUNK), jnp.float32),
        ],
    )(sqk, sqk, sv, sv, spos, spos)


def _comb_body(o0_ref, o1_ref, w0_ref, x1_ref, wo_ref, y1_ref):
    w0 = w0_ref[...]
    oc = o0_ref[...] * w0 + o1_ref[...] * (1.0 - w0)
    y1_ref[...] = x1_ref[...] + jnp.dot(
        oc, wo_ref[...], preferred_element_type=jnp.float32)


def _combine(o0, o1, w0, x1, wo):
    nsb = S // SB
    return pl.pallas_call(
        _comb_body,
        grid=(nsb,),
        in_specs=[
            pl.BlockSpec((SB, D), lambda i: (i, 0)),
            pl.BlockSpec((SB, D), lambda i: (i, 0)),
            pl.BlockSpec((SB, D), lambda i: (i, 0)),
            pl.BlockSpec((SB, D), lambda i: (i, 0)),
            pl.BlockSpec((D, D), lambda i: (0, 0)),
        ],
        out_specs=pl.BlockSpec((SB, D), lambda i: (i, 0)),
        out_shape=jax.ShapeDtypeStruct((S, D), jnp.float32),
    )(o0, o1, w0, x1, wo)


def _ffn_body(y1_ref, x2_ref, g_ref, b_ref, w1_ref, b1_ref, w2_ref, b2_ref,
              y2_ref):
    x = y1_ref[...]
    m = jnp.mean(x, axis=-1, keepdims=True)
    var = jnp.mean((x - m) ** 2, axis=-1, keepdims=True)
    n2 = (x - m) * jax.lax.rsqrt(var + 1e-5) * g_ref[...] + b_ref[...]
    h = jnp.dot(n2, w1_ref[...], preferred_element_type=jnp.float32) + b1_ref[...]
    h = jax.nn.gelu(h)
    y2_ref[...] = x2_ref[...] + jnp.dot(
        h, w2_ref[...], preferred_element_type=jnp.float32) + b2_ref[...]


def _ffn(y1, x2, g, b, w1, b1, w2, b2):
    nsb = S // FSB
    return pl.pallas_call(
        _ffn_body,
        grid=(nsb,),
        in_specs=[
            pl.BlockSpec((FSB, D), lambda i: (i, 0)),
            pl.BlockSpec((FSB, D), lambda i: (i, 0)),
            pl.BlockSpec((1, D), lambda i: (0, 0)),
            pl.BlockSpec((1, D), lambda i: (0, 0)),
            pl.BlockSpec((D, DFF), lambda i: (0, 0)),
            pl.BlockSpec((1, DFF), lambda i: (0, 0)),
            pl.BlockSpec((DFF, D), lambda i: (0, 0)),
            pl.BlockSpec((1, D), lambda i: (0, 0)),
        ],
        out_specs=pl.BlockSpec((FSB, D), lambda i: (i, 0)),
        out_shape=jax.ShapeDtypeStruct((S, D), jnp.float32),
    )(y1, x2, g.reshape(1, D), b.reshape(1, D), w1, b1.reshape(1, DFF), w2,
      b2.reshape(1, D))


def kernel(x1, x2, Wqk, Wv, Wo, ln1_g, ln1_b, ln2_g, ln2_b, W1, b1, W2, b2,
           rot):
    x1 = x1[0]
    x2 = x2[0]
    pos = jnp.arange(S, dtype=jnp.int32)
    for i in range(NLAYERS):
        qk, vv = _proj(x2, ln1_g[i], ln1_b[i], Wqk[i], Wv[i])
        qkh = qk.reshape(S, H, DH).transpose(1, 0, 2)
        vvh = vv.reshape(S, H, DH).transpose(1, 0, 2)
        rotated = jnp.einsum('hsd,drn->hsrn', qkh, rot[i])
        buckets = jnp.argmax(
            jnp.concatenate([rotated, -rotated], axis=-1), axis=-1)
        bkt = buckets.astype(jnp.int32).transpose(2, 0, 1)
        skey = bkt * S + pos[None, None, :]
        sidx = jnp.argsort(skey, axis=-1).astype(jnp.int32)
        sidx_f = sidx.reshape(RH, S)
        rowix = jnp.arange(RH, dtype=jnp.int32)[:, None]
        undo_f = jnp.zeros((RH, S), jnp.int32).at[rowix, sidx_f].set(
            pos[None, :], mode='drop', unique_indices=True)
        qkh2 = jnp.broadcast_to(qkh[None], (R, H, S, DH)).reshape(RH, S, DH)
        vvh2 = jnp.broadcast_to(vvh[None], (R, H, S, DH)).reshape(RH, S, DH)
        sqk = jnp.take_along_axis(qkh2, sidx_f[..., None], axis=1)
        sv = jnp.take_along_axis(vvh2, sidx_f[..., None], axis=1)
        spos = sidx_f.reshape(RH, NC, 1, CHUNK)
        o_s, lse_s = _attention(sqk, sv, spos)
        o_u = jnp.take_along_axis(o_s, undo_f[..., None], axis=1)
        lse_u = jnp.take_along_axis(lse_s.reshape(RH, S), undo_f, axis=1)
        o_u = o_u.reshape(R, H, S, DH).transpose(0, 2, 1, 3).reshape(R, S, D)
        lse_u = lse_u.reshape(R, H, S).transpose(0, 2, 1)
        w = jax.nn.softmax(lse_u, axis=0)
        w0 = jnp.repeat(w[0], DH, axis=-1)
        y1 = _combine(o_u[0], o_u[1], w0, x1, Wo[i])
        y2 = _ffn(y1, x2, ln2_g[i], ln2_b[i], W1[i], b1[i], W2[i], b2[i])
        x1, x2 = y1, y2
    return x2[None]


# speedup vs baseline: 2.5381x; 1.5473x over previous
import functools

import jax
import jax.numpy as jnp
from jax import lax
from jax.experimental import pallas as pl
from jax.experimental.pallas import tpu as pltpu
from jax.experimental.pallas import tpu_sc as plsc

S, D, H, DH = 8192, 768, 12, 64
NLAYERS, R, NB, DFF = 2, 2, 64, 3072
CHUNK = 128
NC = S // CHUNK
RH = R * H
SB = 512
FSB = 256
PW = 2 * DH

_SC_INFO = plsc.get_sparse_core_info()
_NW = _SC_INFO.num_cores * _SC_INFO.num_subcores
GC = 512
N_ROWS = RH * S


def _sc_gather_body(tab, idx_hbm, out, idx_v, rows_v, sem):
    wid = lax.axis_index("s") * _SC_INFO.num_cores + lax.axis_index("c")
    rows_per_w = N_ROWS // _NW
    for i in range(rows_per_w // GC):
        base = wid * rows_per_w + i * GC
        pltpu.sync_copy(idx_hbm.at[pl.ds(base, GC)], idx_v)
        pltpu.async_copy(tab.at[idx_v], rows_v, sem).wait()
        pltpu.sync_copy(rows_v, out.at[pl.ds(base, GC)])


def _sc_gather(tab, gidx):
    mesh = plsc.VectorSubcoreMesh(core_axis_name="c", subcore_axis_name="s")
    f = pl.kernel(
        _sc_gather_body,
        mesh=mesh,
        out_type=jax.ShapeDtypeStruct((N_ROWS, PW), jnp.float32),
        scratch_types=[pltpu.VMEM((GC,), jnp.int32),
                       pltpu.VMEM((GC, PW), jnp.float32),
                       pltpu.SemaphoreType.DMA],
    )
    return f(tab, gidx)


def _proj_body(x_ref, g_ref, b_ref, wqk_ref, wv_ref, rot_ref, qkv_ref,
               rotf_ref):
    x = x_ref[...]
    m = jnp.mean(x, axis=-1, keepdims=True)
    var = jnp.mean((x - m) ** 2, axis=-1, keepdims=True)
    n = (x - m) * jax.lax.rsqrt(var + 1e-5) * g_ref[...] + b_ref[...]
    qk = jnp.dot(n, wqk_ref[...], preferred_element_type=jnp.float32)
    v = jnp.dot(n, wv_ref[...], preferred_element_type=jnp.float32)
    rot2 = rot_ref[...]
    for h in range(H):
        qk_h = qk[:, h * DH:(h + 1) * DH]
        qkv_ref[h, :, 0:DH] = qk_h
        qkv_ref[h, :, DH:PW] = v[:, h * DH:(h + 1) * DH]
        rotf_ref[h] = jnp.dot(qk_h, rot2, preferred_element_type=jnp.float32)


def _proj(x, g, b, wqk, wv, rot2):
    nsb = S // SB
    return pl.pallas_call(
        _proj_body,
        grid=(nsb,),
        in_specs=[
            pl.BlockSpec((SB, D), lambda i: (i, 0)),
            pl.BlockSpec((1, D), lambda i: (0, 0)),
            pl.BlockSpec((1, D), lambda i: (0, 0)),
            pl.BlockSpec((D, D), lambda i: (0, 0)),
            pl.BlockSpec((D, D), lambda i: (0, 0)),
            pl.BlockSpec((DH, DH), lambda i: (0, 0)),
        ],
        out_specs=[
            pl.BlockSpec((H, SB, PW), lambda i: (0, i, 0)),
            pl.BlockSpec((H, SB, DH), lambda i: (0, i, 0)),
        ],
        out_shape=[
            jax.ShapeDtypeStruct((H, S, PW), jnp.float32),
            jax.ShapeDtypeStruct((H, S, DH), jnp.float32),
        ],
    )(x, g.reshape(1, D), b.reshape(1, D), wqk, wv, rot2)


NCP = 16
SEG = NCP * CHUNK


def _attn_one_chunk(q, kp, vc, vp, pq, pp, o_ref, k):
    k2 = jnp.concatenate([kp, q], axis=0)
    v2 = jnp.concatenate([vp, vc], axis=0)
    norm = jnp.sqrt(jnp.sum(k2 * k2, axis=-1, keepdims=True))
    k2n = k2 / jnp.maximum(norm, 1e-6)
    dots = jax.lax.dot_general(
        q, k2n, (((1,), (1,)), ((), ())),
        preferred_element_type=jnp.float32) * (DH ** -0.5)
    pk = jnp.concatenate([pp, pq], axis=1)
    pqc = jnp.transpose(pq)
    dots = jnp.where(pqc >= pk, dots, -1e9)
    dots = jnp.where(pqc == pk, dots - 1e5, dots)
    m = jnp.max(dots, axis=-1, keepdims=True)
    ex = jnp.exp(dots - m)
    ssum = jnp.sum(ex, axis=-1, keepdims=True)
    rows = slice(k * CHUNK, (k + 1) * CHUNK)
    o_ref[0, rows, 0:DH] = jnp.dot(
        ex / ssum, v2, preferred_element_type=jnp.float32)
    o_ref[0, rows, DH:DH + 1] = m + jnp.log(ssum)
    o_ref[0, rows, DH + 1:PW] = jnp.zeros((CHUNK, DH - 1), jnp.float32)


def _attn_body(cur_ref, prevb_ref, pcur_ref, pprevb_ref, o_ref):
    for k in range(NCP):
        rows = slice(k * CHUNK, (k + 1) * CHUNK)
        q = cur_ref[0, rows, 0:DH]
        vc = cur_ref[0, rows, DH:PW]
        if k == 0:
            kp = prevb_ref[0, :, 0:DH]
            vp = prevb_ref[0, :, DH:PW]
            pp = pprevb_ref[0, 0]
        else:
            prows = slice((k - 1) * CHUNK, k * CHUNK)
            kp = cur_ref[0, prows, 0:DH]
            vp = cur_ref[0, prows, DH:PW]
            pp = pcur_ref[0, k - 1, :, :]
        pq = pcur_ref[0, k, :, :]
        _attn_one_chunk(q, kp, vc, vp, pq, pp, o_ref, k)


def _attention(sqkv, spos):
    prevc = lambda cb: jax.lax.rem(cb * NCP + NC - 1, NC)
    return pl.pallas_call(
        _attn_body,
        grid=(RH, S // SEG),
        in_specs=[
            pl.BlockSpec((1, SEG, PW), lambda j, cb: (j, cb, 0)),
            pl.BlockSpec((1, CHUNK, PW), lambda j, cb: (j, prevc(cb), 0)),
            pl.BlockSpec((1, NCP, 1, CHUNK), lambda j, cb: (j, cb, 0, 0)),
            pl.BlockSpec((1, 1, 1, CHUNK), lambda j, cb: (j, prevc(cb), 0, 0)),
        ],
        out_specs=pl.BlockSpec((1, SEG, PW), lambda j, cb: (j, cb, 0)),
        out_shape=jax.ShapeDtypeStruct((RH, S, PW), jnp.float32),
    )(sqkv, sqkv, spos, spos)


def _comb_body(o0_ref, o1_ref, w0_ref, x1_ref, wo_ref, y1_ref):
    w0 = w0_ref[...]
    oc = o0_ref[...] * w0 + o1_ref[...] * (1.0 - w0)
    y1_ref[...] = x1_ref[...] + jnp.dot(
        oc, wo_ref[...], preferred_element_type=jnp.float32)


def _combine(o0, o1, w0, x1, wo):
    nsb = S // SB
    return pl.pallas_call(
        _comb_body,
        grid=(nsb,),
        in_specs=[
            pl.BlockSpec((SB, D), lambda i: (i, 0)),
            pl.BlockSpec((SB, D), lambda i: (i, 0)),
            pl.BlockSpec((SB, D), lambda i: (i, 0)),
            pl.BlockSpec((SB, D), lambda i: (i, 0)),
            pl.BlockSpec((D, D), lambda i: (0, 0)),
        ],
        out_specs=pl.BlockSpec((SB, D), lambda i: (i, 0)),
        out_shape=jax.ShapeDtypeStruct((S, D), jnp.float32),
    )(o0, o1, w0, x1, wo)


def _ffn_body(y1_ref, x2_ref, g_ref, b_ref, w1_ref, b1_ref, w2_ref, b2_ref,
              y2_ref):
    x = y1_ref[...]
    m = jnp.mean(x, axis=-1, keepdims=True)
    var = jnp.mean((x - m) ** 2, axis=-1, keepdims=True)
    n2 = (x - m) * jax.lax.rsqrt(var + 1e-5) * g_ref[...] + b_ref[...]
    h = jnp.dot(n2, w1_ref[...], preferred_element_type=jnp.float32) + b1_ref[...]
    h = jax.nn.gelu(h)
    y2_ref[...] = x2_ref[...] + jnp.dot(
        h, w2_ref[...], preferred_element_type=jnp.float32) + b2_ref[...]


def _ffn(y1, x2, g, b, w1, b1, w2, b2):
    nsb = S // FSB
    return pl.pallas_call(
        _ffn_body,
        grid=(nsb,),
        in_specs=[
            pl.BlockSpec((FSB, D), lambda i: (i, 0)),
            pl.BlockSpec((FSB, D), lambda i: (i, 0)),
            pl.BlockSpec((1, D), lambda i: (0, 0)),
            pl.BlockSpec((1, D), lambda i: (0, 0)),
            pl.BlockSpec((D, DFF), lambda i: (0, 0)),
            pl.BlockSpec((1, DFF), lambda i: (0, 0)),
            pl.BlockSpec((DFF, D), lambda i: (0, 0)),
            pl.BlockSpec((1, D), lambda i: (0, 0)),
        ],
        out_specs=pl.BlockSpec((FSB, D), lambda i: (i, 0)),
        out_shape=jax.ShapeDtypeStruct((S, D), jnp.float32),
    )(y1, x2, g.reshape(1, D), b.reshape(1, D), w1, b1.reshape(1, DFF), w2,
      b2.reshape(1, D))


def kernel(x1, x2, Wqk, Wv, Wo, ln1_g, ln1_b, ln2_g, ln2_b, W1, b1, W2, b2,
           rot):
    x1 = x1[0]
    x2 = x2[0]
    pos = jnp.arange(S, dtype=jnp.int32)
    rowix = jnp.arange(RH, dtype=jnp.int32)[:, None]
    for i in range(NLAYERS):
        rot2 = rot[i].reshape(DH, R * (NB // 2))
        qkv_t, rotf = _proj(x2, ln1_g[i], ln1_b[i], Wqk[i], Wv[i], rot2)
        rotated = rotf.reshape(H, S, R, NB // 2)
        buckets = jnp.argmax(
            jnp.concatenate([rotated, -rotated], axis=-1), axis=-1)
        bkt = buckets.astype(jnp.int32).transpose(2, 0, 1)
        skey = bkt * S + pos[None, None, :]
        sidx = jnp.argsort(skey, axis=-1).astype(jnp.int32)
        sidx_f = sidx.reshape(RH, S)
        undo_f = jnp.argsort(sidx_f, axis=-1).astype(jnp.int32)
        gidx = ((rowix % H) * S + sidx_f).reshape(N_ROWS)
        sqkv = _sc_gather(qkv_t.reshape(H * S, PW), gidx).reshape(RH, S, PW)
        spos = sidx_f.reshape(RH, NC, 1, CHUNK)
        oaug_s = _attention(sqkv, spos)
        ugidx = (rowix * S + undo_f).reshape(N_ROWS)
        oaug_u = _sc_gather(oaug_s.reshape(N_ROWS, PW), ugidx)
        oaug_u = oaug_u.reshape(R, H, S, PW)
        o_u = oaug_u[..., 0:DH].transpose(0, 2, 1, 3).reshape(R, S, D)
        lse_u = oaug_u[..., DH].transpose(0, 2, 1)
        w = jax.nn.softmax(lse_u, axis=0)
        w0 = jnp.repeat(w[0], DH, axis=-1)
        y1 = _combine(o_u[0], o_u[1], w0, x1, Wo[i])
        y2 = _ffn(y1, x2, ln2_g[i], ln2_b[i], W1[i], b1[i], W2[i], b2[i])
        x1, x2 = y1, y2
    return x2[None]


# speedup vs baseline: 2.5470x; 1.0035x over previous
import functools

import jax
import jax.numpy as jnp
from jax import lax
from jax.experimental import pallas as pl
from jax.experimental.pallas import tpu as pltpu
from jax.experimental.pallas import tpu_sc as plsc

S, D, H, DH = 8192, 768, 12, 64
NLAYERS, R, NB, DFF = 2, 2, 64, 3072
CHUNK = 128
NC = S // CHUNK
RH = R * H
SB = 512
FSB = 256
PW = 2 * DH

_SC_INFO = plsc.get_sparse_core_info()
_NW = _SC_INFO.num_cores * _SC_INFO.num_subcores
GC = 256
N_ROWS = RH * S


def _sc_gather_body(tab, idx_hbm, out, idx_v0, idx_v1, rows_v0, rows_v1,
                    sem0, sem1):
    wid = lax.axis_index("s") * _SC_INFO.num_cores + lax.axis_index("c")
    rows_per_w = N_ROWS // _NW
    nst = rows_per_w // GC
    base = wid * rows_per_w
    idx_v = (idx_v0, idx_v1)
    rows_v = (rows_v0, rows_v1)
    sems = (sem0, sem1)
    pltpu.sync_copy(idx_hbm.at[pl.ds(base, GC)], idx_v0)
    inflight = pltpu.async_copy(tab.at[idx_v0], rows_v0, sem0)
    for i in range(nst):
        cur = i % 2
        nxt = (i + 1) % 2
        if i + 1 < nst:
            pltpu.sync_copy(idx_hbm.at[pl.ds(base + (i + 1) * GC, GC)],
                            idx_v[nxt])
            nxt_copy = pltpu.async_copy(tab.at[idx_v[nxt]], rows_v[nxt],
                                        sems[nxt])
        inflight.wait()
        pltpu.sync_copy(rows_v[cur], out.at[pl.ds(base + i * GC, GC)])
        if i + 1 < nst:
            inflight = nxt_copy


def _sc_gather(tab, gidx):
    mesh = plsc.VectorSubcoreMesh(core_axis_name="c", subcore_axis_name="s")
    f = pl.kernel(
        _sc_gather_body,
        mesh=mesh,
        out_type=jax.ShapeDtypeStruct((N_ROWS, PW), jnp.float32),
        scratch_types=[pltpu.VMEM((GC,), jnp.int32),
                       pltpu.VMEM((GC,), jnp.int32),
                       pltpu.VMEM((GC, PW), jnp.float32),
                       pltpu.VMEM((GC, PW), jnp.float32),
                       pltpu.SemaphoreType.DMA,
                       pltpu.SemaphoreType.DMA],
    )
    return f(tab, gidx)


def _proj_body(x_ref, g_ref, b_ref, wqk_ref, wv_ref, rot_ref, qkv_ref,
               rotf_ref):
    x = x_ref[...]
    m = jnp.mean(x, axis=-1, keepdims=True)
    var = jnp.mean((x - m) ** 2, axis=-1, keepdims=True)
    n = (x - m) * jax.lax.rsqrt(var + 1e-5) * g_ref[...] + b_ref[...]
    qk = jnp.dot(n, wqk_ref[...], preferred_element_type=jnp.float32)
    v = jnp.dot(n, wv_ref[...], preferred_element_type=jnp.float32)
    rot2 = rot_ref[...]
    for h in range(H):
        qk_h = qk[:, h * DH:(h + 1) * DH]
        qkv_ref[h, :, 0:DH] = qk_h
        qkv_ref[h, :, DH:PW] = v[:, h * DH:(h + 1) * DH]
        rotf_ref[h] = jnp.dot(qk_h, rot2, preferred_element_type=jnp.float32)


def _proj(x, g, b, wqk, wv, rot2):
    nsb = S // SB
    return pl.pallas_call(
        _proj_body,
        grid=(nsb,),
        in_specs=[
            pl.BlockSpec((SB, D), lambda i: (i, 0)),
            pl.BlockSpec((1, D), lambda i: (0, 0)),
            pl.BlockSpec((1, D), lambda i: (0, 0)),
            pl.BlockSpec((D, D), lambda i: (0, 0)),
            pl.BlockSpec((D, D), lambda i: (0, 0)),
            pl.BlockSpec((DH, DH), lambda i: (0, 0)),
        ],
        out_specs=[
            pl.BlockSpec((H, SB, PW), lambda i: (0, i, 0)),
            pl.BlockSpec((H, SB, DH), lambda i: (0, i, 0)),
        ],
        out_shape=[
            jax.ShapeDtypeStruct((H, S, PW), jnp.float32),
            jax.ShapeDtypeStruct((H, S, DH), jnp.float32),
        ],
    )(x, g.reshape(1, D), b.reshape(1, D), wqk, wv, rot2)


NCP = 16
SEG = NCP * CHUNK


def _attn_one_chunk(q, kp, vc, vp, pq, pp, o_ref, k):
    k2 = jnp.concatenate([kp, q], axis=0)
    v2 = jnp.concatenate([vp, vc], axis=0)
    norm = jnp.sqrt(jnp.sum(k2 * k2, axis=-1, keepdims=True))
    k2n = k2 / jnp.maximum(norm, 1e-6)
    dots = jax.lax.dot_general(
        q, k2n, (((1,), (1,)), ((), ())),
        preferred_element_type=jnp.float32) * (DH ** -0.5)
    pk = jnp.concatenate([pp, pq], axis=1)
    pqc = jnp.transpose(pq)
    dots = jnp.where(pqc >= pk, dots, -1e9)
    dots = jnp.where(pqc == pk, dots - 1e5, dots)
    m = jnp.max(dots, axis=-1, keepdims=True)
    ex = jnp.exp(dots - m)
    ssum = jnp.sum(ex, axis=-1, keepdims=True)
    rows = slice(k * CHUNK, (k + 1) * CHUNK)
    o_ref[0, rows, 0:DH] = jnp.dot(
        ex / ssum, v2, preferred_element_type=jnp.float32)
    o_ref[0, rows, DH:DH + 1] = m + jnp.log(ssum)
    o_ref[0, rows, DH + 1:PW] = jnp.zeros((CHUNK, DH - 1), jnp.float32)


def _attn_body(cur_ref, prevb_ref, pcur_ref, pprevb_ref, o_ref):
    for k in range(NCP):
        rows = slice(k * CHUNK, (k + 1) * CHUNK)
        q = cur_ref[0, rows, 0:DH]
        vc = cur_ref[0, rows, DH:PW]
        if k == 0:
            kp = prevb_ref[0, :, 0:DH]
            vp = prevb_ref[0, :, DH:PW]
            pp = pprevb_ref[0, 0]
        else:
            prows = slice((k - 1) * CHUNK, k * CHUNK)
            kp = cur_ref[0, prows, 0:DH]
            vp = cur_ref[0, prows, DH:PW]
            pp = pcur_ref[0, k - 1, :, :]
        pq = pcur_ref[0, k, :, :]
        _attn_one_chunk(q, kp, vc, vp, pq, pp, o_ref, k)


def _attention(sqkv, spos):
    prevc = lambda cb: jax.lax.rem(cb * NCP + NC - 1, NC)
    return pl.pallas_call(
        _attn_body,
        grid=(RH, S // SEG),
        in_specs=[
            pl.BlockSpec((1, SEG, PW), lambda j, cb: (j, cb, 0)),
            pl.BlockSpec((1, CHUNK, PW), lambda j, cb: (j, prevc(cb), 0)),
            pl.BlockSpec((1, NCP, 1, CHUNK), lambda j, cb: (j, cb, 0, 0)),
            pl.BlockSpec((1, 1, 1, CHUNK), lambda j, cb: (j, prevc(cb), 0, 0)),
        ],
        out_specs=pl.BlockSpec((1, SEG, PW), lambda j, cb: (j, cb, 0)),
        out_shape=jax.ShapeDtypeStruct((RH, S, PW), jnp.float32),
    )(sqkv, sqkv, spos, spos)


def _comb_body(o0_ref, o1_ref, w0_ref, x1_ref, wo_ref, y1_ref):
    w0 = w0_ref[...]
    oc = o0_ref[...] * w0 + o1_ref[...] * (1.0 - w0)
    y1_ref[...] = x1_ref[...] + jnp.dot(
        oc, wo_ref[...], preferred_element_type=jnp.float32)


def _combine(o0, o1, w0, x1, wo):
    nsb = S // SB
    return pl.pallas_call(
        _comb_body,
        grid=(nsb,),
        in_specs=[
            pl.BlockSpec((SB, D), lambda i: (i, 0)),
            pl.BlockSpec((SB, D), lambda i: (i, 0)),
            pl.BlockSpec((SB, D), lambda i: (i, 0)),
            pl.BlockSpec((SB, D), lambda i: (i, 0)),
            pl.BlockSpec((D, D), lambda i: (0, 0)),
        ],
        out_specs=pl.BlockSpec((SB, D), lambda i: (i, 0)),
        out_shape=jax.ShapeDtypeStruct((S, D), jnp.float32),
    )(o0, o1, w0, x1, wo)


def _ffn_body(y1_ref, x2_ref, g_ref, b_ref, w1_ref, b1_ref, w2_ref, b2_ref,
              y2_ref):
    x = y1_ref[...]
    m = jnp.mean(x, axis=-1, keepdims=True)
    var = jnp.mean((x - m) ** 2, axis=-1, keepdims=True)
    n2 = (x - m) * jax.lax.rsqrt(var + 1e-5) * g_ref[...] + b_ref[...]
    h = jnp.dot(n2, w1_ref[...], preferred_element_type=jnp.float32) + b1_ref[...]
    h = jax.nn.gelu(h)
    y2_ref[...] = x2_ref[...] + jnp.dot(
        h, w2_ref[...], preferred_element_type=jnp.float32) + b2_ref[...]


def _ffn(y1, x2, g, b, w1, b1, w2, b2):
    nsb = S // FSB
    return pl.pallas_call(
        _ffn_body,
        grid=(nsb,),
        in_specs=[
            pl.BlockSpec((FSB, D), lambda i: (i, 0)),
            pl.BlockSpec((FSB, D), lambda i: (i, 0)),
            pl.BlockSpec((1, D), lambda i: (0, 0)),
            pl.BlockSpec((1, D), lambda i: (0, 0)),
            pl.BlockSpec((D, DFF), lambda i: (0, 0)),
            pl.BlockSpec((1, DFF), lambda i: (0, 0)),
            pl.BlockSpec((DFF, D), lambda i: (0, 0)),
            pl.BlockSpec((1, D), lambda i: (0, 0)),
        ],
        out_specs=pl.BlockSpec((FSB, D), lambda i: (i, 0)),
        out_shape=jax.ShapeDtypeStruct((S, D), jnp.float32),
    )(y1, x2, g.reshape(1, D), b.reshape(1, D), w1, b1.reshape(1, DFF), w2,
      b2.reshape(1, D))


def kernel(x1, x2, Wqk, Wv, Wo, ln1_g, ln1_b, ln2_g, ln2_b, W1, b1, W2, b2,
           rot):
    x1 = x1[0]
    x2 = x2[0]
    pos = jnp.arange(S, dtype=jnp.int32)
    rowix = jnp.arange(RH, dtype=jnp.int32)[:, None]
    for i in range(NLAYERS):
        rot2 = rot[i].reshape(DH, R * (NB // 2))
        qkv_t, rotf = _proj(x2, ln1_g[i], ln1_b[i], Wqk[i], Wv[i], rot2)
        rotated = rotf.reshape(H, S, R, NB // 2)
        buckets = jnp.argmax(
            jnp.concatenate([rotated, -rotated], axis=-1), axis=-1)
        bkt = buckets.astype(jnp.int32).transpose(2, 0, 1)
        skey = bkt * S + pos[None, None, :]
        sidx = jnp.argsort(skey, axis=-1).astype(jnp.int32)
        sidx_f = sidx.reshape(RH, S)
        undo_f = jnp.argsort(sidx_f, axis=-1).astype(jnp.int32)
        gidx = ((rowix % H) * S + sidx_f).reshape(N_ROWS)
        sqkv = _sc_gather(qkv_t.reshape(H * S, PW), gidx).reshape(RH, S, PW)
        spos = sidx_f.reshape(RH, NC, 1, CHUNK)
        oaug_s = _attention(sqkv, spos)
        ugidx = (rowix * S + undo_f).reshape(N_ROWS)
        oaug_u = _sc_gather(oaug_s.reshape(N_ROWS, PW), ugidx)
        oaug_u = oaug_u.reshape(R, H, S, PW)
        o_u = oaug_u[..., 0:DH].transpose(0, 2, 1, 3).reshape(R, S, D)
        lse_u = oaug_u[..., DH].transpose(0, 2, 1)
        w = jax.nn.softmax(lse_u, axis=0)
        w0 = jnp.repeat(w[0], DH, axis=-1)
        y1 = _combine(o_u[0], o_u[1], w0, x1, Wo[i])
        y2 = _ffn(y1, x2, ln2_g[i], ln2_b[i], W1[i], b1[i], W2[i], b2[i])
        x1, x2 = y1, y2
    return x2[None]


# speedup vs baseline: 2.9716x; 1.1667x over previous
import functools

import jax
import jax.numpy as jnp
from jax import lax
from jax.experimental import pallas as pl
from jax.experimental.pallas import tpu as pltpu
from jax.experimental.pallas import tpu_sc as plsc

S, D, H, DH = 8192, 768, 12, 64
NLAYERS, R, NB, DFF = 2, 2, 64, 3072
CHUNK = 128
NC = S // CHUNK
RH = R * H
SB = 512
FSB = 256
PW = 2 * DH

_SC_INFO = plsc.get_sparse_core_info()
_NW = _SC_INFO.num_cores * _SC_INFO.num_subcores
GC = 256
N_ROWS = RH * S


def _sc_gather_body(tab, idx_hbm, out, idx_v0, idx_v1, rows_v0, rows_v1,
                    sem0, sem1):
    wid = lax.axis_index("s") * _SC_INFO.num_cores + lax.axis_index("c")
    rows_per_w = N_ROWS // _NW
    nst = rows_per_w // GC
    base = wid * rows_per_w
    idx_v = (idx_v0, idx_v1)
    rows_v = (rows_v0, rows_v1)
    sems = (sem0, sem1)
    pltpu.sync_copy(idx_hbm.at[pl.ds(base, GC)], idx_v0)
    inflight = pltpu.async_copy(tab.at[idx_v0], rows_v0, sem0)
    for i in range(nst):
        cur = i % 2
        nxt = (i + 1) % 2
        if i + 1 < nst:
            pltpu.sync_copy(idx_hbm.at[pl.ds(base + (i + 1) * GC, GC)],
                            idx_v[nxt])
            nxt_copy = pltpu.async_copy(tab.at[idx_v[nxt]], rows_v[nxt],
                                        sems[nxt])
        inflight.wait()
        pltpu.sync_copy(rows_v[cur], out.at[pl.ds(base + i * GC, GC)])
        if i + 1 < nst:
            inflight = nxt_copy


def _sc_gather(tab, gidx):
    mesh = plsc.VectorSubcoreMesh(core_axis_name="c", subcore_axis_name="s")
    f = pl.kernel(
        _sc_gather_body,
        mesh=mesh,
        out_type=jax.ShapeDtypeStruct((N_ROWS, PW), jnp.float32),
        scratch_types=[pltpu.VMEM((GC,), jnp.int32),
                       pltpu.VMEM((GC,), jnp.int32),
                       pltpu.VMEM((GC, PW), jnp.float32),
                       pltpu.VMEM((GC, PW), jnp.float32),
                       pltpu.SemaphoreType.DMA,
                       pltpu.SemaphoreType.DMA],
    )
    return f(tab, gidx)


def _proj_body(x_ref, g_ref, b_ref, wqk_ref, wv_ref, rot_ref, qkv_ref,
               rotf_ref):
    x = x_ref[...]
    m = jnp.mean(x, axis=-1, keepdims=True)
    var = jnp.mean((x - m) ** 2, axis=-1, keepdims=True)
    n = (x - m) * jax.lax.rsqrt(var + 1e-5) * g_ref[...] + b_ref[...]
    qk = jnp.dot(n, wqk_ref[...], preferred_element_type=jnp.float32)
    v = jnp.dot(n, wv_ref[...], preferred_element_type=jnp.float32)
    rot2 = rot_ref[...]
    for h in range(H):
        qk_h = qk[:, h * DH:(h + 1) * DH]
        qkv_ref[h, :, 0:DH] = qk_h
        qkv_ref[h, :, DH:PW] = v[:, h * DH:(h + 1) * DH]
        rotf_ref[h] = jnp.dot(qk_h, rot2, preferred_element_type=jnp.float32)


def _proj(x, g, b, wqk, wv, rot2):
    nsb = S // SB
    return pl.pallas_call(
        _proj_body,
        grid=(nsb,),
        in_specs=[
            pl.BlockSpec((SB, D), lambda i: (i, 0)),
            pl.BlockSpec((1, D), lambda i: (0, 0)),
            pl.BlockSpec((1, D), lambda i: (0, 0)),
            pl.BlockSpec((D, D), lambda i: (0, 0)),
            pl.BlockSpec((D, D), lambda i: (0, 0)),
            pl.BlockSpec((DH, DH), lambda i: (0, 0)),
        ],
        out_specs=[
            pl.BlockSpec((H, SB, PW), lambda i: (0, i, 0)),
            pl.BlockSpec((H, SB, DH), lambda i: (0, i, 0)),
        ],
        out_shape=[
            jax.ShapeDtypeStruct((H, S, PW), jnp.float32),
            jax.ShapeDtypeStruct((H, S, DH), jnp.float32),
        ],
    )(x, g.reshape(1, D), b.reshape(1, D), wqk, wv, rot2)


NCP = 16
SEG = NCP * CHUNK


def _attn_one_chunk(q, kp, vc, vp, pq, pp, o_ref, k):
    k2 = jnp.concatenate([kp, q], axis=0)
    v2 = jnp.concatenate([vp, vc], axis=0)
    norm = jnp.sqrt(jnp.sum(k2 * k2, axis=-1, keepdims=True))
    k2n = k2 / jnp.maximum(norm, 1e-6)
    dots = jax.lax.dot_general(
        q, k2n, (((1,), (1,)), ((), ())),
        preferred_element_type=jnp.float32) * (DH ** -0.5)
    pk = jnp.concatenate([pp, pq], axis=1)
    pqc = jnp.transpose(pq)
    dots = jnp.where(pqc >= pk, dots, -1e9)
    dots = jnp.where(pqc == pk, dots - 1e5, dots)
    m = jnp.max(dots, axis=-1, keepdims=True)
    ex = jnp.exp(dots - m)
    ssum = jnp.sum(ex, axis=-1, keepdims=True)
    rows = slice(k * CHUNK, (k + 1) * CHUNK)
    o_ref[0, rows, 0:DH] = jnp.dot(
        ex / ssum, v2, preferred_element_type=jnp.float32)
    o_ref[0, rows, DH:DH + 1] = m + jnp.log(ssum)
    o_ref[0, rows, DH + 1:PW] = jnp.zeros((CHUNK, DH - 1), jnp.float32)


def _attn_body(cur_ref, prevb_ref, pcur_ref, pprevb_ref, o_ref):
    for k in range(NCP):
        rows = slice(k * CHUNK, (k + 1) * CHUNK)
        q = cur_ref[0, rows, 0:DH]
        vc = cur_ref[0, rows, DH:PW]
        if k == 0:
            kp = prevb_ref[0, :, 0:DH]
            vp = prevb_ref[0, :, DH:PW]
            pp = pprevb_ref[0, 0]
        else:
            prows = slice((k - 1) * CHUNK, k * CHUNK)
            kp = cur_ref[0, prows, 0:DH]
            vp = cur_ref[0, prows, DH:PW]
            pp = pcur_ref[0, k - 1, :, :]
        pq = pcur_ref[0, k, :, :]
        _attn_one_chunk(q, kp, vc, vp, pq, pp, o_ref, k)


def _attention(sqkv, spos):
    prevc = lambda cb: jax.lax.rem(cb * NCP + NC - 1, NC)
    return pl.pallas_call(
        _attn_body,
        grid=(RH, S // SEG),
        in_specs=[
            pl.BlockSpec((1, SEG, PW), lambda j, cb: (j, cb, 0)),
            pl.BlockSpec((1, CHUNK, PW), lambda j, cb: (j, prevc(cb), 0)),
            pl.BlockSpec((1, NCP, 1, CHUNK), lambda j, cb: (j, cb, 0, 0)),
            pl.BlockSpec((1, 1, 1, CHUNK), lambda j, cb: (j, prevc(cb), 0, 0)),
        ],
        out_specs=pl.BlockSpec((1, SEG, PW), lambda j, cb: (j, cb, 0)),
        out_shape=jax.ShapeDtypeStruct((RH, S, PW), jnp.float32),
    )(sqkv, sqkv, spos, spos)


def _comb_body(aug_ref, x1_ref, wo_ref, y1_ref):
    acc = x1_ref[...]
    for h in range(H):
        o0 = aug_ref[h, :, 0:DH]
        o1 = aug_ref[H + h, :, 0:DH]
        lse0 = aug_ref[h, :, DH:DH + 1]
        lse1 = aug_ref[H + h, :, DH:DH + 1]
        w0 = jax.nn.sigmoid(lse0 - lse1)
        oc = o0 * w0 + o1 * (1.0 - w0)
        acc = acc + jnp.dot(oc, wo_ref[h * DH:(h + 1) * DH, :],
                            preferred_element_type=jnp.float32)
    y1_ref[...] = acc


def _combine(aug, x1, wo):
    nsb = S // SB
    return pl.pallas_call(
        _comb_body,
        grid=(nsb,),
        in_specs=[
            pl.BlockSpec((RH, SB, PW), lambda i: (0, i, 0)),
            pl.BlockSpec((SB, D), lambda i: (i, 0)),
            pl.BlockSpec((D, D), lambda i: (0, 0)),
        ],
        out_specs=pl.BlockSpec((SB, D), lambda i: (i, 0)),
        out_shape=jax.ShapeDtypeStruct((S, D), jnp.float32),
    )(aug, x1, wo)


def _ffn_body(y1_ref, x2_ref, g_ref, b_ref, w1_ref, b1_ref, w2_ref, b2_ref,
              y2_ref):
    x = y1_ref[...]
    m = jnp.mean(x, axis=-1, keepdims=True)
    var = jnp.mean((x - m) ** 2, axis=-1, keepdims=True)
    n2 = (x - m) * jax.lax.rsqrt(var + 1e-5) * g_ref[...] + b_ref[...]
    h = jnp.dot(n2, w1_ref[...], preferred_element_type=jnp.float32) + b1_ref[...]
    h = jax.nn.gelu(h)
    y2_ref[...] = x2_ref[...] + jnp.dot(
        h, w2_ref[...], preferred_element_type=jnp.float32) + b2_ref[...]


def _ffn(y1, x2, g, b, w1, b1, w2, b2):
    nsb = S // FSB
    return pl.pallas_call(
        _ffn_body,
        grid=(nsb,),
        in_specs=[
            pl.BlockSpec((FSB, D), lambda i: (i, 0)),
            pl.BlockSpec((FSB, D), lambda i: (i, 0)),
            pl.BlockSpec((1, D), lambda i: (0, 0)),
            pl.BlockSpec((1, D), lambda i: (0, 0)),
            pl.BlockSpec((D, DFF), lambda i: (0, 0)),
            pl.BlockSpec((1, DFF), lambda i: (0, 0)),
            pl.BlockSpec((DFF, D), lambda i: (0, 0)),
            pl.BlockSpec((1, D), lambda i: (0, 0)),
        ],
        out_specs=pl.BlockSpec((FSB, D), lambda i: (i, 0)),
        out_shape=jax.ShapeDtypeStruct((S, D), jnp.float32),
    )(y1, x2, g.reshape(1, D), b.reshape(1, D), w1, b1.reshape(1, DFF), w2,
      b2.reshape(1, D))


def kernel(x1, x2, Wqk, Wv, Wo, ln1_g, ln1_b, ln2_g, ln2_b, W1, b1, W2, b2,
           rot):
    x1 = x1[0]
    x2 = x2[0]
    pos = jnp.arange(S, dtype=jnp.int32)
    rowix = jnp.arange(RH, dtype=jnp.int32)[:, None]
    for i in range(NLAYERS):
        rot2 = rot[i].reshape(DH, R * (NB // 2))
        qkv_t, rotf = _proj(x2, ln1_g[i], ln1_b[i], Wqk[i], Wv[i], rot2)
        rotated = rotf.reshape(H, S, R, NB // 2)
        buckets = jnp.argmax(
            jnp.concatenate([rotated, -rotated], axis=-1), axis=-1)
        bkt = buckets.astype(jnp.int32).transpose(2, 0, 1)
        skey = bkt * S + pos[None, None, :]
        sidx = jnp.argsort(skey, axis=-1).astype(jnp.int32)
        sidx_f = sidx.reshape(RH, S)
        undo_f = jnp.argsort(sidx_f, axis=-1).astype(jnp.int32)
        gidx = ((rowix % H) * S + sidx_f).reshape(N_ROWS)
        sqkv = _sc_gather(qkv_t.reshape(H * S, PW), gidx).reshape(RH, S, PW)
        spos = sidx_f.reshape(RH, NC, 1, CHUNK)
        oaug_s = _attention(sqkv, spos)
        ugidx = (rowix * S + undo_f).reshape(N_ROWS)
        oaug_u = _sc_gather(oaug_s.reshape(N_ROWS, PW), ugidx)
        y1 = _combine(oaug_u.reshape(RH, S, PW), x1, Wo[i])
        y2 = _ffn(y1, x2, ln2_g[i], ln2_b[i], W1[i], b1[i], W2[i], b2[i])
        x1, x2 = y1, y2
    return x2[None]
